# baseline (speedup 1.0000x reference)
import functools

import jax
import jax.numpy as jnp
from jax import lax
from jax.experimental import pallas as pl
from jax.experimental.pallas import tpu as pltpu
from jax.experimental.pallas import tpu_sc as plsc

M = 300
NUM_CLASSES = 80
BOX_DIM = 4
OUT_W = NUM_CLASSES * BOX_DIM
STRIPE = 8
NUM_WORKERS = 32
FULL_STRIPES = M // STRIPE
TAIL_ROWS = M - FULL_STRIPES * STRIPE
LANES = 16
CHUNKS = OUT_W // LANES


def _body(t_hbm, w_hbm, labels_hbm, out_t_hbm, out_w_hbm,
          labels_v, mask_tab, t_v, w_v, out_t_v, out_w_v,
          sem_lab, sem_t, sem_w, sem_out):
    wid = lax.axis_index("s") * 2 + lax.axis_index("c")

    iota = lax.broadcasted_iota(jnp.int32, (LANES,), 0)
    iota4 = iota & 3
    zeros = jnp.zeros((LANES,), jnp.float32)
    ones = jnp.ones((LANES,), jnp.float32)

    pltpu.async_copy(labels_hbm, labels_v, sem_lab).wait()
    for i in range(NUM_CLASSES // LANES):
        mask_tab[pl.ds(i * LANES, LANES)] = zeros
    lab_vec = plsc.load_gather(labels_v, [iota & 7])
    plsc.store_scatter(mask_tab, [lab_vec], ones)

    m = [plsc.load_gather(mask_tab, [(i * LANES + iota) >> 2])
         for i in range(CHUNKS)]

    def stripe(base, nrows):
        rows = pl.ds(0, nrows)
        t_cp = pltpu.async_copy(t_hbm.at[pl.ds(base, nrows)],
                                t_v.at[rows], sem_t)
        w_cp = pltpu.async_copy(w_hbm.at[pl.ds(base, nrows)],
                                w_v.at[rows], sem_w)
        t_cp.wait()
        w_cp.wait()

        def row_body(r, carry):
            r16 = jnp.full((LANES,), r, jnp.int32)
            t_row = plsc.load_gather(t_v, [r16, iota4])
            for v in range(CHUNKS):
                out_t_v[r, pl.ds(v * LANES, LANES)] = m[v] * t_row
            for v in range(CHUNKS):
                out_w_v[r, pl.ds(v * LANES, LANES)] = zeros
            rg16 = r16 + base
            r_eff16 = jnp.minimum(rg16, NUM_CLASSES - 1)
            w_row = plsc.load_gather(w_v, [r16, iota4])
            mval = plsc.load_gather(mask_tab, [r_eff16])
            col = r_eff16 * BOX_DIM + iota4
            lane_mask = (iota < BOX_DIM) & (rg16 < NUM_CLASSES)
            plsc.store_scatter(out_w_v, [r16, col], w_row * mval,
                               mask=lane_mask)
            return carry

        lax.fori_loop(0, nrows, row_body, 0)

        t_out = pltpu.async_copy(out_t_v.at[rows],
                                 out_t_hbm.at[pl.ds(base, nrows)], sem_out)
        w_out = pltpu.async_copy(out_w_v.at[rows],
                                 out_w_hbm.at[pl.ds(base, nrows)], sem_out)
        t_out.wait()
        w_out.wait()

    stripe(wid * STRIPE, STRIPE)

    @pl.when(wid < FULL_STRIPES - NUM_WORKERS)
    def _():
        stripe((NUM_WORKERS + wid) * STRIPE, STRIPE)

    @pl.when(wid == FULL_STRIPES - NUM_WORKERS)
    def _():
        stripe(FULL_STRIPES * STRIPE, TAIL_ROWS)


@jax.jit
def kernel(bbox_targets, bbox_weights, labels):
    mesh = plsc.VectorSubcoreMesh(core_axis_name="c", subcore_axis_name="s")
    return pl.kernel(
        _body,
        out_type=(jax.ShapeDtypeStruct((M, OUT_W), jnp.float32),
                  jax.ShapeDtypeStruct((M, OUT_W), jnp.float32)),
        mesh=mesh,
        compiler_params=pltpu.CompilerParams(use_tc_tiling_on_sc=True,
                                             needs_layout_passes=False,
                                             skip_device_barrier=True),
        scratch_types=[
            pltpu.VMEM((8,), jnp.int32),
            pltpu.VMEM((NUM_CLASSES,), jnp.float32),
            pltpu.VMEM((STRIPE, BOX_DIM), jnp.float32),
            pltpu.VMEM((STRIPE, BOX_DIM), jnp.float32),
            pltpu.VMEM((STRIPE, OUT_W), jnp.float32),
            pltpu.VMEM((STRIPE, OUT_W), jnp.float32),
            pltpu.SemaphoreType.DMA,
            pltpu.SemaphoreType.DMA,
            pltpu.SemaphoreType.DMA,
            pltpu.SemaphoreType.DMA,
        ],
    )(bbox_targets, bbox_weights, labels)


# speedup vs baseline: 1.1334x; 1.1334x over previous
import functools

import jax
import jax.numpy as jnp
from jax import lax
from jax.experimental import pallas as pl
from jax.experimental.pallas import tpu as pltpu
from jax.experimental.pallas import tpu_sc as plsc

M = 300
NUM_CLASSES = 80
BOX_DIM = 4
OUT_W = NUM_CLASSES * BOX_DIM
STRIPE = 24
FULL_WORKERS = M // STRIPE
TAIL8_BASE = FULL_WORKERS * STRIPE
TAIL4_BASE = TAIL8_BASE + 8
LANES = 16
CHUNKS = OUT_W // LANES


def _body(t_hbm, w_hbm, labels_hbm, out_t_hbm, out_w_hbm,
          labels_v, mask_tab, t_v, w_v, out_t_v, out_w_v,
          sem_lab, sem_t, sem_w, sem_out):
    wid = lax.axis_index("s")
    iota = lax.broadcasted_iota(jnp.int32, (LANES,), 0)
    iota4 = iota & 3
    zeros = jnp.zeros((LANES,), jnp.float32)
    ones = jnp.ones((LANES,), jnp.float32)

    lab_cp = pltpu.async_copy(labels_hbm, labels_v, sem_lab)

    def start_inputs(base, nrows):
        rows = pl.ds(0, nrows)
        pltpu.async_copy(t_hbm.at[pl.ds(base, nrows)], t_v.at[rows], sem_t)
        pltpu.async_copy(w_hbm.at[pl.ds(base, nrows)], w_v.at[rows], sem_w)

    @pl.when(wid < FULL_WORKERS)
    def _():
        start_inputs(wid * STRIPE, STRIPE)

    @pl.when(wid == FULL_WORKERS)
    def _():
        start_inputs(TAIL8_BASE, 8)

    @pl.when(wid == FULL_WORKERS + 1)
    def _():
        start_inputs(TAIL4_BASE, 4)

    for i in range(NUM_CLASSES // LANES):
        mask_tab[pl.ds(i * LANES, LANES)] = zeros
    lab_cp.wait()
    lab_vec = plsc.load_gather(labels_v, [iota & 7])
    plsc.store_scatter(mask_tab, [lab_vec], ones)

    m = [plsc.load_gather(mask_tab, [(i * LANES + iota) >> 2])
         for i in range(CHUNKS)]

    def process(base, nrows):
        rows = pl.ds(0, nrows)
        pltpu.make_async_copy(t_hbm.at[pl.ds(base, nrows)],
                              t_v.at[rows], sem_t).wait()
        pltpu.make_async_copy(w_hbm.at[pl.ds(base, nrows)],
                              w_v.at[rows], sem_w).wait()

        def row_body(r, carry):
            r16 = jnp.full((LANES,), r, jnp.int32)
            t_row = plsc.load_gather(t_v, [r16, iota4])
            for v in range(CHUNKS):
                out_t_v[r, pl.ds(v * LANES, LANES)] = m[v] * t_row
            for v in range(CHUNKS):
                out_w_v[r, pl.ds(v * LANES, LANES)] = zeros
            rg16 = r16 + base
            r_eff16 = jnp.minimum(rg16, NUM_CLASSES - 1)
            w_row = plsc.load_gather(w_v, [r16, iota4])
            mval = plsc.load_gather(mask_tab, [r_eff16])
            col = r_eff16 * BOX_DIM + iota4
            lane_mask = (iota < BOX_DIM) & (rg16 < NUM_CLASSES)
            plsc.store_scatter(out_w_v, [r16, col], w_row * mval,
                               mask=lane_mask)
            return carry

        lax.fori_loop(0, nrows, row_body, 0)

        t_out = pltpu.async_copy(out_t_v.at[rows],
                                 out_t_hbm.at[pl.ds(base, nrows)], sem_out)
        w_out = pltpu.async_copy(out_w_v.at[rows],
                                 out_w_hbm.at[pl.ds(base, nrows)], sem_out)
        t_out.wait()
        w_out.wait()

    @pl.when(wid < FULL_WORKERS)
    def _():
        process(wid * STRIPE, STRIPE)

    @pl.when(wid == FULL_WORKERS)
    def _():
        process(TAIL8_BASE, 8)

    @pl.when(wid == FULL_WORKERS + 1)
    def _():
        process(TAIL4_BASE, 4)


@jax.jit
def kernel(bbox_targets, bbox_weights, labels):
    mesh = plsc.VectorSubcoreMesh(core_axis_name="c", subcore_axis_name="s",
                                  num_cores=1)
    return pl.kernel(
        _body,
        out_type=(jax.ShapeDtypeStruct((M, OUT_W), jnp.float32),
                  jax.ShapeDtypeStruct((M, OUT_W), jnp.float32)),
        mesh=mesh,
        compiler_params=pltpu.CompilerParams(use_tc_tiling_on_sc=True,
                                             needs_layout_passes=False,
                                             skip_device_barrier=True),
        scratch_types=[
            pltpu.VMEM((8,), jnp.int32),
            pltpu.VMEM((NUM_CLASSES,), jnp.float32),
            pltpu.VMEM((STRIPE, BOX_DIM), jnp.float32),
            pltpu.VMEM((STRIPE, BOX_DIM), jnp.float32),
            pltpu.VMEM((STRIPE, OUT_W), jnp.float32),
            pltpu.VMEM((STRIPE, OUT_W), jnp.float32),
            pltpu.SemaphoreType.DMA,
            pltpu.SemaphoreType.DMA,
            pltpu.SemaphoreType.DMA,
            pltpu.SemaphoreType.DMA,
        ],
    )(bbox_targets, bbox_weights, labels)


# speedup vs baseline: 1.2063x; 1.0644x over previous
import functools

import jax
import jax.numpy as jnp
from jax import lax
from jax.experimental import pallas as pl
from jax.experimental.pallas import tpu as pltpu
from jax.experimental.pallas import tpu_sc as plsc

M = 300
NUM_CLASSES = 80
BOX_DIM = 4
OUT_W = NUM_CLASSES * BOX_DIM
STRIPE = 24
FULL_WORKERS = M // STRIPE
TAIL8_BASE = FULL_WORKERS * STRIPE
TAIL4_BASE = TAIL8_BASE + 8
LANES = 16
CHUNKS = OUT_W // LANES
NUM_LABELS = 8


def _sc_body(w_hbm, labels_hbm, out_w_hbm,
             labels_v, mask_tab, w_v, out_w_v,
             sem_lab, sem_w, sem_out):
    wid = lax.axis_index("s")
    iota = lax.broadcasted_iota(jnp.int32, (LANES,), 0)
    iota4 = iota & 3
    zeros = jnp.zeros((LANES,), jnp.float32)
    ones = jnp.ones((LANES,), jnp.float32)

    lab_cp = pltpu.async_copy(labels_hbm, labels_v, sem_lab)

    def start_inputs(base, nrows):
        pltpu.async_copy(w_hbm.at[pl.ds(base, nrows)],
                         w_v.at[pl.ds(0, nrows)], sem_w)

    @pl.when(wid < FULL_WORKERS)
    def _():
        start_inputs(wid * STRIPE, STRIPE)

    @pl.when(wid == FULL_WORKERS)
    def _():
        start_inputs(TAIL8_BASE, 8)

    @pl.when(wid == FULL_WORKERS + 1)
    def _():
        start_inputs(TAIL4_BASE, 4)

    for i in range(NUM_CLASSES // LANES):
        mask_tab[pl.ds(i * LANES, LANES)] = zeros
    lab_cp.wait()
    lab_vec = plsc.load_gather(labels_v, [iota & 7])
    plsc.store_scatter(mask_tab, [lab_vec], ones)

    def process(base, nrows):
        rows = pl.ds(0, nrows)
        pltpu.make_async_copy(w_hbm.at[pl.ds(base, nrows)],
                              w_v.at[rows], sem_w).wait()

        def row_body(r, carry):
            r16 = jnp.full((LANES,), r, jnp.int32)
            for v in range(CHUNKS):
                out_w_v[r, pl.ds(v * LANES, LANES)] = zeros
            rg16 = r16 + base
            r_eff16 = jnp.minimum(rg16, NUM_CLASSES - 1)
            w_row = plsc.load_gather(w_v, [r16, iota4])
            mval = plsc.load_gather(mask_tab, [r_eff16])
            col = r_eff16 * BOX_DIM + iota4
            lane_mask = (iota < BOX_DIM) & (rg16 < NUM_CLASSES)
            plsc.store_scatter(out_w_v, [r16, col], w_row * mval,
                               mask=lane_mask)
            return carry

        lax.fori_loop(0, nrows, row_body, 0)

        pltpu.async_copy(out_w_v.at[rows],
                         out_w_hbm.at[pl.ds(base, nrows)], sem_out).wait()

    @pl.when(wid < FULL_WORKERS)
    def _():
        process(wid * STRIPE, STRIPE)

    @pl.when(wid == FULL_WORKERS)
    def _():
        process(TAIL8_BASE, 8)

    @pl.when(wid == FULL_WORKERS + 1)
    def _():
        process(TAIL4_BASE, 4)


def _tc_body(labels_smem, t_ref, out_ref):
    col = lax.broadcasted_iota(jnp.int32, (1, OUT_W), 1)
    cls = col >> 2
    box = col & 3
    mask = cls == labels_smem[0]
    for k in range(1, NUM_LABELS):
        mask = mask | (cls == labels_smem[k])
    t = t_ref[...]
    tt = jnp.where(
        box == 0, t[:, 0:1],
        jnp.where(box == 1, t[:, 1:2],
                  jnp.where(box == 2, t[:, 2:3], t[:, 3:4])))
    out_ref[...] = jnp.where(mask, tt, jnp.float32(0.0))


@jax.jit
def kernel(bbox_targets, bbox_weights, labels):
    out_t = pl.pallas_call(
        _tc_body,
        out_shape=jax.ShapeDtypeStruct((M, OUT_W), jnp.float32),
        in_specs=[pl.BlockSpec(memory_space=pltpu.SMEM),
                  pl.BlockSpec(memory_space=pltpu.VMEM)],
        out_specs=pl.BlockSpec(memory_space=pltpu.VMEM),
    )(labels, bbox_targets)

    mesh = plsc.VectorSubcoreMesh(core_axis_name="c", subcore_axis_name="s",
                                  num_cores=1)
    out_w = pl.kernel(
        _sc_body,
        out_type=jax.ShapeDtypeStruct((M, OUT_W), jnp.float32),
        mesh=mesh,
        compiler_params=pltpu.CompilerParams(use_tc_tiling_on_sc=True,
                                             needs_layout_passes=False,
                                             skip_device_barrier=True),
        scratch_types=[
            pltpu.VMEM((NUM_LABELS,), jnp.int32),
            pltpu.VMEM((NUM_CLASSES,), jnp.float32),
            pltpu.VMEM((STRIPE, BOX_DIM), jnp.float32),
            pltpu.VMEM((STRIPE, OUT_W), jnp.float32),
            pltpu.SemaphoreType.DMA,
            pltpu.SemaphoreType.DMA,
            pltpu.SemaphoreType.DMA,
        ],
    )(bbox_weights, labels)
    return (out_t, out_w)
